# baseline (speedup 1.0000x reference)
import functools

import jax
import jax.numpy as jnp
from jax import lax
from jax.experimental import pallas as pl
from jax.experimental.pallas import tpu as pltpu
from jax.experimental.pallas import tpu_sc as plsc

N = 10000
E = 320000
D = 128
G = 64

_INFO = plsc.get_sparse_core_info()
NC = _INFO.num_cores
NS = _INFO.num_subcores
NW = NC * NS
CHUNK = 128
E_PAD = 327680
NCHUNK_TOT = E_PAD // CHUNK
CPS = NCHUNK_TOT // NS
C0 = 128
BLK0 = 64
NST0 = C0 // BLK0
BLK1 = 32
NST1 = (CPS - C0) // BLK1
BLKMAX = max(BLK0, BLK1)
N_ACC = 10112
RPT = N_ACC // NS


NBUF = 2


def _seg_sum_body(h0_hbm, h1_hbm, src_hbm, dst_hbm, zero_hbm, out_hbm,
                  src_v, dst_v, r0, r1, agg_sh, s0, s1):
    bufs = (r0, r1)
    sems = (s0, s1)
    c = lax.axis_index("c")
    s = lax.axis_index("s")

    pltpu.sync_copy(zero_hbm.at[pl.ds(s * RPT, RPT)],
                    agg_sh.at[pl.ds(s * RPT, RPT)])
    plsc.subcore_barrier()

    def run_stages(h_hbm, nstages, blk, base):
        for st in range(nstages):
            start = base + st * blk
            pltpu.sync_copy(src_hbm.at[pl.ds(start, blk)],
                            src_v.at[pl.ds(0, blk)])
            pltpu.sync_copy(dst_hbm.at[pl.ds(start, blk)],
                            dst_v.at[pl.ds(0, blk)])
            for b in range(NBUF):
                pltpu.async_copy(h_hbm.at[src_v.at[b]], bufs[b], sems[b])

            def body(g, _):
                for b in range(NBUF):
                    j = g * NBUF + b
                    pltpu.make_async_copy(h_hbm.at[src_v.at[j]], bufs[b],
                                          sems[b]).wait()
                    pltpu.sync_copy(bufs[b], agg_sh.at[dst_v.at[j]],
                                    add=True)

                    @pl.when(j + NBUF < blk)
                    def _prefetch():
                        pltpu.async_copy(h_hbm.at[src_v.at[j + NBUF]],
                                         bufs[b], sems[b])
                return _

            lax.fori_loop(0, blk // NBUF, body, None)

    @pl.when(c == 0)
    def _core0():
        run_stages(h0_hbm, NST0, BLK0, s * CPS)

    @pl.when(c == 1)
    def _core1():
        run_stages(h1_hbm, NST1, BLK1, s * CPS + C0)

    plsc.subcore_barrier()

    pltpu.sync_copy(agg_sh.at[pl.ds(s * RPT, RPT)],
                    out_hbm.at[c, pl.ds(s * RPT, RPT)])


_seg_sum = functools.partial(
    pl.kernel,
    mesh=plsc.VectorSubcoreMesh(core_axis_name="c", subcore_axis_name="s"),
    out_type=jax.ShapeDtypeStruct((NC, N_ACC, D), jnp.float32),
    scratch_types=[
        pltpu.VMEM((BLKMAX, CHUNK), jnp.int32),
        pltpu.VMEM((BLKMAX, CHUNK), jnp.int32),
        pltpu.VMEM((CHUNK, D), jnp.float32),
        pltpu.VMEM((CHUNK, D), jnp.float32),
        pltpu.VMEM_SHARED((N_ACC, D), jnp.float32),
        pltpu.SemaphoreType.DMA,
        pltpu.SemaphoreType.DMA,
    ],
)(_seg_sum_body)


def _dense_layer_body(h_ref, p_ref, w1_ref, b1_ref, w2_ref, b2_ref,
                      g_ref, beta_ref, o_ref, o2_ref, *, apply_ln):
    z = h_ref[...] + p_ref[0] + p_ref[1]
    z = jnp.maximum(jnp.dot(z, w1_ref[...],
                            preferred_element_type=jnp.float32) + b1_ref[...], 0.0)
    z = jnp.dot(z, w2_ref[...], preferred_element_type=jnp.float32) + b2_ref[...]
    h = jnp.maximum(z, 0.0)
    if apply_ln:
        mu = jnp.mean(h, axis=-1, keepdims=True)
        var = jnp.mean((h - mu) ** 2, axis=-1, keepdims=True)
        h = (h - mu) / jnp.sqrt(var + 1e-5) * g_ref[...] + beta_ref[...]
    o_ref[...] = h
    o2_ref[...] = h


def _dense_layer(h, p, w1, b1, w2, b2, g, beta, apply_ln):
    blk = 1000
    grid = N // blk
    return pl.pallas_call(
        functools.partial(_dense_layer_body, apply_ln=apply_ln),
        grid=(grid,),
        in_specs=[
            pl.BlockSpec((blk, D), lambda i: (i, 0)),
            pl.BlockSpec((NC, blk, D), lambda i: (0, i, 0)),
            pl.BlockSpec((D, D), lambda i: (0, 0)),
            pl.BlockSpec((1, D), lambda i: (0, 0)),
            pl.BlockSpec((D, D), lambda i: (0, 0)),
            pl.BlockSpec((1, D), lambda i: (0, 0)),
            pl.BlockSpec((1, D), lambda i: (0, 0)),
            pl.BlockSpec((1, D), lambda i: (0, 0)),
        ],
        out_specs=[pl.BlockSpec((blk, D), lambda i: (i, 0)),
                   pl.BlockSpec((blk, D), lambda i: (i, 0))],
        out_shape=[jax.ShapeDtypeStruct((N, D), jnp.float32),
                   jax.ShapeDtypeStruct((N, D), jnp.float32)],
    )(h, p, w1, b1.reshape(1, D), w2, b2.reshape(1, D),
      g.reshape(1, D), beta.reshape(1, D))


def _dup_body(x_ref, o_ref, o2_ref):
    v = x_ref[...]
    o_ref[...] = v
    o2_ref[...] = v


def _dup(x):
    blk = 1000
    return pl.pallas_call(
        _dup_body,
        grid=(N // blk,),
        in_specs=[pl.BlockSpec((blk, D), lambda i: (i, 0))],
        out_specs=[pl.BlockSpec((blk, D), lambda i: (i, 0)),
                   pl.BlockSpec((blk, D), lambda i: (i, 0))],
        out_shape=[jax.ShapeDtypeStruct((N, D), jnp.float32),
                   jax.ShapeDtypeStruct((N, D), jnp.float32)],
    )(x)


def _pool_head_body(h_ref, batch_ref, wp1_ref, bp1_ref, wp2_ref, bp2_ref,
                    emb_ref, out_ref):
    gids = lax.broadcasted_iota(jnp.int32, (N, G), 1)
    onehot = jnp.where(batch_ref[...] == gids, 1.0, 0.0)
    pooled = lax.dot_general(onehot, h_ref[...], (((0,), (0,)), ((), ())),
                             preferred_element_type=jnp.float32)
    counts = jnp.sum(onehot, axis=0).reshape(G, 1)
    pooled = pooled / jnp.maximum(counts, 1.0)
    z = jnp.dot(pooled, wp1_ref[...],
                preferred_element_type=jnp.float32) + bp1_ref[...]
    z = jnp.dot(z, wp2_ref[...],
                preferred_element_type=jnp.float32) + bp2_ref[...]
    emb_ref[...] = z
    m = jnp.max(z, axis=1, keepdims=True)
    lse = m + jnp.log(jnp.sum(jnp.exp(z - m), axis=1, keepdims=True))
    out_ref[...] = z - lse


def _pool_head(h, batch, wp1, bp1, wp2, bp2):
    return pl.pallas_call(
        _pool_head_body,
        out_shape=(jax.ShapeDtypeStruct((G, 2), jnp.float32),
                   jax.ShapeDtypeStruct((G, 2), jnp.float32)),
    )(h, batch.reshape(N, 1), wp1, bp1.reshape(1, D), wp2, bp2.reshape(1, 2))


@jax.jit
def kernel(x, edge_index, batch, params):
    src = edge_index[0]
    dst = edge_index[1]
    pad = E_PAD - E
    srcp = jnp.concatenate([src, jnp.zeros((pad,), jnp.int32)]
                           ).reshape(NCHUNK_TOT, CHUNK)
    dstp = jnp.concatenate([dst, jnp.full((pad,), N, jnp.int32)]
                           ).reshape(NCHUNK_TOT, CHUNK)
    zero = jnp.zeros((N_ACC, D), jnp.float32)

    h, h2 = _dup(x)
    for i in range(3):
        parts = _seg_sum(h, h2, srcp, dstp, zero)
        h, h2 = _dense_layer(h, parts,
                         params[f"W1_{i}"], params[f"b1_{i}"],
                         params[f"W2_{i}"], params[f"b2_{i}"],
                         params[f"ln_g_{i}"] if i != 2 else params["ln_g_0"],
                         params[f"ln_b_{i}"] if i != 2 else params["ln_b_0"],
                         apply_ln=(i != 2))
    emb, out = _pool_head(h, batch, params["Wp1"], params["bp1"],
                          params["Wp2"], params["bp2"])
    return (emb, out)


# speedup vs baseline: 1.1404x; 1.1404x over previous
import functools

import jax
import jax.numpy as jnp
from jax import lax
from jax.experimental import pallas as pl
from jax.experimental.pallas import tpu as pltpu
from jax.experimental.pallas import tpu_sc as plsc

N = 10000
E = 320000
D = 128
G = 64

_INFO = plsc.get_sparse_core_info()
NC = _INFO.num_cores
NS = _INFO.num_subcores
NW = NC * NS
CHUNK = 128
E_PAD = 327680
NCHUNK_TOT = E_PAD // CHUNK
CPS = NCHUNK_TOT // NS
C0 = 120
BLK0 = 40
NST0 = C0 // BLK0
BLK1 = 40
NST1 = (CPS - C0) // BLK1
BLKMAX = max(BLK0, BLK1)
N_ACC = 10112
RPT = N_ACC // NS


NBUF = 2


def _seg_sum_body(h0_hbm, h1_hbm, src_hbm, dst_hbm, zero_hbm, out_hbm,
                  src_v, dst_v, r0, r1, agg_sh, s0, s1):
    bufs = (r0, r1)
    sems = (s0, s1)
    c = lax.axis_index("c")
    s = lax.axis_index("s")

    pltpu.sync_copy(zero_hbm.at[pl.ds(s * RPT, RPT)],
                    agg_sh.at[pl.ds(s * RPT, RPT)])
    plsc.subcore_barrier()

    def run_stages(h_hbm, nstages, blk, base):
        for st in range(nstages):
            start = base + st * blk
            pltpu.sync_copy(src_hbm.at[pl.ds(start, blk)],
                            src_v.at[pl.ds(0, blk)])
            pltpu.sync_copy(dst_hbm.at[pl.ds(start, blk)],
                            dst_v.at[pl.ds(0, blk)])
            for b in range(NBUF):
                pltpu.async_copy(h_hbm.at[src_v.at[b]], bufs[b], sems[b])

            def body(g, _):
                for b in range(NBUF):
                    j = g * NBUF + b
                    pltpu.make_async_copy(h_hbm.at[src_v.at[j]], bufs[b],
                                          sems[b]).wait()
                    pltpu.sync_copy(bufs[b], agg_sh.at[dst_v.at[j]],
                                    add=True)

                    @pl.when(j + NBUF < blk)
                    def _prefetch():
                        pltpu.async_copy(h_hbm.at[src_v.at[j + NBUF]],
                                         bufs[b], sems[b])
                return _

            lax.fori_loop(0, blk // NBUF, body, None)

    @pl.when(c == 0)
    def _core0():
        run_stages(h0_hbm, NST0, BLK0, s * CPS)

    @pl.when(c == 1)
    def _core1():
        run_stages(h1_hbm, NST1, BLK1, s * CPS + C0)

    plsc.subcore_barrier()

    pltpu.sync_copy(agg_sh.at[pl.ds(s * RPT, RPT)],
                    out_hbm.at[c, pl.ds(s * RPT, RPT)])


_seg_sum = functools.partial(
    pl.kernel,
    mesh=plsc.VectorSubcoreMesh(core_axis_name="c", subcore_axis_name="s"),
    out_type=jax.ShapeDtypeStruct((NC, N_ACC, D), jnp.float32),
    scratch_types=[
        pltpu.VMEM((BLKMAX, CHUNK), jnp.int32),
        pltpu.VMEM((BLKMAX, CHUNK), jnp.int32),
        pltpu.VMEM((CHUNK, D), jnp.float32),
        pltpu.VMEM((CHUNK, D), jnp.float32),
        pltpu.VMEM_SHARED((N_ACC, D), jnp.float32),
        pltpu.SemaphoreType.DMA,
        pltpu.SemaphoreType.DMA,
    ],
)(_seg_sum_body)


def _dense_layer_body(h_ref, p_ref, w1_ref, b1_ref, w2_ref, b2_ref,
                      g_ref, beta_ref, o_ref, o2_ref, *, apply_ln):
    z = h_ref[...] + p_ref[0] + p_ref[1]
    z = jnp.maximum(jnp.dot(z, w1_ref[...],
                            preferred_element_type=jnp.float32) + b1_ref[...], 0.0)
    z = jnp.dot(z, w2_ref[...], preferred_element_type=jnp.float32) + b2_ref[...]
    h = jnp.maximum(z, 0.0)
    if apply_ln:
        mu = jnp.mean(h, axis=-1, keepdims=True)
        var = jnp.mean((h - mu) ** 2, axis=-1, keepdims=True)
        h = (h - mu) / jnp.sqrt(var + 1e-5) * g_ref[...] + beta_ref[...]
    o_ref[...] = h
    o2_ref[...] = h


def _dense_layer(h, p, w1, b1, w2, b2, g, beta, apply_ln):
    blk = 1000
    grid = N // blk
    return pl.pallas_call(
        functools.partial(_dense_layer_body, apply_ln=apply_ln),
        grid=(grid,),
        in_specs=[
            pl.BlockSpec((blk, D), lambda i: (i, 0)),
            pl.BlockSpec((NC, blk, D), lambda i: (0, i, 0)),
            pl.BlockSpec((D, D), lambda i: (0, 0)),
            pl.BlockSpec((1, D), lambda i: (0, 0)),
            pl.BlockSpec((D, D), lambda i: (0, 0)),
            pl.BlockSpec((1, D), lambda i: (0, 0)),
            pl.BlockSpec((1, D), lambda i: (0, 0)),
            pl.BlockSpec((1, D), lambda i: (0, 0)),
        ],
        out_specs=[pl.BlockSpec((blk, D), lambda i: (i, 0)),
                   pl.BlockSpec((blk, D), lambda i: (i, 0))],
        out_shape=[jax.ShapeDtypeStruct((N, D), jnp.float32),
                   jax.ShapeDtypeStruct((N, D), jnp.float32)],
    )(h, p, w1, b1.reshape(1, D), w2, b2.reshape(1, D),
      g.reshape(1, D), beta.reshape(1, D))


def _dup_body(x_ref, o_ref, o2_ref):
    v = x_ref[...]
    o_ref[...] = v
    o2_ref[...] = v


def _dup(x):
    blk = 1000
    return pl.pallas_call(
        _dup_body,
        grid=(N // blk,),
        in_specs=[pl.BlockSpec((blk, D), lambda i: (i, 0))],
        out_specs=[pl.BlockSpec((blk, D), lambda i: (i, 0)),
                   pl.BlockSpec((blk, D), lambda i: (i, 0))],
        out_shape=[jax.ShapeDtypeStruct((N, D), jnp.float32),
                   jax.ShapeDtypeStruct((N, D), jnp.float32)],
    )(x)


def _pool_head_body(h_ref, batch_ref, wp1_ref, bp1_ref, wp2_ref, bp2_ref,
                    emb_ref, out_ref):
    gids = lax.broadcasted_iota(jnp.int32, (N, G), 1)
    onehot = jnp.where(batch_ref[...] == gids, 1.0, 0.0)
    pooled = lax.dot_general(onehot, h_ref[...], (((0,), (0,)), ((), ())),
                             preferred_element_type=jnp.float32)
    counts = jnp.sum(onehot, axis=0).reshape(G, 1)
    pooled = pooled / jnp.maximum(counts, 1.0)
    z = jnp.dot(pooled, wp1_ref[...],
                preferred_element_type=jnp.float32) + bp1_ref[...]
    z = jnp.dot(z, wp2_ref[...],
                preferred_element_type=jnp.float32) + bp2_ref[...]
    emb_ref[...] = z
    m = jnp.max(z, axis=1, keepdims=True)
    lse = m + jnp.log(jnp.sum(jnp.exp(z - m), axis=1, keepdims=True))
    out_ref[...] = z - lse


def _pool_head(h, batch, wp1, bp1, wp2, bp2):
    return pl.pallas_call(
        _pool_head_body,
        out_shape=(jax.ShapeDtypeStruct((G, 2), jnp.float32),
                   jax.ShapeDtypeStruct((G, 2), jnp.float32)),
    )(h, batch.reshape(N, 1), wp1, bp1.reshape(1, D), wp2, bp2.reshape(1, 2))


@jax.jit
def kernel(x, edge_index, batch, params):
    src = edge_index[0]
    dst = edge_index[1]
    pad = E_PAD - E
    srcp = jnp.concatenate([src, jnp.zeros((pad,), jnp.int32)]
                           ).reshape(NCHUNK_TOT, CHUNK)
    dstp = jnp.concatenate([dst, jnp.full((pad,), N, jnp.int32)]
                           ).reshape(NCHUNK_TOT, CHUNK)
    zero = jnp.zeros((N_ACC, D), jnp.float32)

    h, h2 = _dup(x)
    for i in range(3):
        parts = _seg_sum(h, h2, srcp, dstp, zero)
        h, h2 = _dense_layer(h, parts,
                         params[f"W1_{i}"], params[f"b1_{i}"],
                         params[f"W2_{i}"], params[f"b2_{i}"],
                         params[f"ln_g_{i}"] if i != 2 else params["ln_g_0"],
                         params[f"ln_b_{i}"] if i != 2 else params["ln_b_0"],
                         apply_ln=(i != 2))
    emb, out = _pool_head(h, batch, params["Wp1"], params["bp1"],
                          params["Wp2"], params["bp2"])
    return (emb, out)
